# baseline (speedup 1.0000x reference)
import functools

import jax
import jax.numpy as jnp
from jax import lax
from jax.experimental import pallas as pl
from jax.experimental.pallas import tpu as pltpu
from jax.experimental.pallas import tpu_sc as plsc

MAX_SEQS = 1024
MAX_TOKENS = 8192
NUM_NEW = 1024

_GROUPS = NUM_NEW // 16
_CHUNKS = NUM_NEW // 128
_ROWS_PER_BLOCK = 128


def _sc_stage1_body(num_tokens_hbm, sids_hbm, cnt_out_hbm, idx_out_hbm,
                    sids_v, cnt_v, idx_v):
  @pl.when((lax.axis_index("c") == 0) & (lax.axis_index("s") == 0))
  def _():
    pltpu.sync_copy(sids_hbm, sids_v)
    pltpu.sync_copy(num_tokens_hbm, cnt_v)

    @pl.loop(0, _GROUPS)
    def _(g):
      v = sids_v[g]
      r, is_last = plsc.scan_count(v)
      base = plsc.load_gather(cnt_v, [v])
      pos = base + r - 1
      plsc.store_scatter(cnt_v, [v], pos + 1, mask=is_last)
      flat = v * MAX_TOKENS + pos
      idx_v[g // 8, pl.ds((g % 8) * 16, 16)] = flat

    pltpu.sync_copy(cnt_v, cnt_out_hbm)
    pltpu.sync_copy(idx_v, idx_out_hbm)


def _sc_stage1(num_tokens, sids):
  mesh = plsc.VectorSubcoreMesh(core_axis_name="c", subcore_axis_name="s")
  return pl.kernel(
      _sc_stage1_body,
      out_type=(
          jax.ShapeDtypeStruct((MAX_SEQS,), jnp.int32),
          jax.ShapeDtypeStruct((_CHUNKS, 128), jnp.int32),
      ),
      mesh=mesh,
      compiler_params=pltpu.CompilerParams(needs_layout_passes=False),
      scratch_types=[
          pltpu.VMEM((_GROUPS, 16), jnp.int32),
          pltpu.VMEM((MAX_SEQS,), jnp.int32),
          pltpu.VMEM((_CHUNKS, 128), jnp.int32),
      ],
  )(num_tokens, sids)


def _tc_copy_body(tok_in, lp_in, tok_out, lp_out):
  tok_out[...] = tok_in[...]
  lp_out[...] = lp_in[...]


def _tc_copy(tokens, logprobs):
  nblk = MAX_SEQS // _ROWS_PER_BLOCK
  spec = pl.BlockSpec((_ROWS_PER_BLOCK, MAX_TOKENS), lambda i: (i, 0))
  return pl.pallas_call(
      _tc_copy_body,
      grid=(nblk,),
      in_specs=[spec, spec],
      out_specs=[spec, spec],
      out_shape=(
          jax.ShapeDtypeStruct((MAX_SEQS, MAX_TOKENS), jnp.int32),
          jax.ShapeDtypeStruct((MAX_SEQS, MAX_TOKENS), jnp.float32),
      ),
      compiler_params=pltpu.CompilerParams(
          dimension_semantics=("arbitrary",),
      ),
  )(tokens, logprobs)


def _sc_stage2_body(idx_hbm, ntok_hbm, nlp_hbm, tok_flat, lp_flat,
                    idx_v, val_i, val_f, sem):
  cid = lax.axis_index("c")
  sid = lax.axis_index("s")

  @pl.when(sid < _CHUNKS)
  def _():
    pltpu.sync_copy(idx_hbm.at[sid], idx_v)

    @pl.when(cid == 0)
    def _():
      pltpu.sync_copy(ntok_hbm.at[sid], val_i)
      pltpu.async_copy(val_i, tok_flat.at[idx_v], sem).wait()

    @pl.when(cid == 1)
    def _():
      pltpu.sync_copy(nlp_hbm.at[sid], val_f)
      pltpu.async_copy(val_f, lp_flat.at[idx_v], sem).wait()


def _sc_stage2(idx, ntok, nlp, tok_ref, lp_ref):
  mesh = plsc.VectorSubcoreMesh(core_axis_name="c", subcore_axis_name="s")
  return pl.kernel(
      _sc_stage2_body,
      out_type=(),
      mesh=mesh,
      compiler_params=pltpu.CompilerParams(needs_layout_passes=False),
      scratch_types=[
          pltpu.VMEM((128,), jnp.int32),
          pltpu.VMEM((128,), jnp.int32),
          pltpu.VMEM((128,), jnp.float32),
          pltpu.SemaphoreType.DMA,
      ],
  )(idx, ntok, nlp, tok_ref, lp_ref)


@jax.jit
def _kernel_impl(tokens, logprobs, num_tokens, local_seq_ids, new_tokens,
                 new_log_probs):
  cnt, idx = _sc_stage1(num_tokens, local_seq_ids.reshape(_GROUPS, 16))
  tok_c, lp_c = _tc_copy(tokens, logprobs)
  tok_ref = jax.new_ref(tok_c.reshape(-1))
  lp_ref = jax.new_ref(lp_c.reshape(-1))
  _sc_stage2(idx, new_tokens.reshape(_CHUNKS, 128),
             new_log_probs.reshape(_CHUNKS, 128), tok_ref, lp_ref)
  return (
      tok_ref[...].reshape(MAX_SEQS, MAX_TOKENS),
      lp_ref[...].reshape(MAX_SEQS, MAX_TOKENS),
      cnt,
  )


def kernel(tokens, logprobs, num_tokens, local_seq_ids, new_tokens,
           new_log_probs, num_new_tokens):
  del num_new_tokens
  return _kernel_impl(tokens, logprobs, num_tokens, local_seq_ids, new_tokens,
                      new_log_probs)


# speedup vs baseline: 1.0236x; 1.0236x over previous
import functools

import jax
import jax.numpy as jnp
from jax import lax
from jax.experimental import pallas as pl
from jax.experimental.pallas import tpu as pltpu
from jax.experimental.pallas import tpu_sc as plsc

MAX_SEQS = 1024
MAX_TOKENS = 8192
NUM_NEW = 1024

_GROUPS = NUM_NEW // 16
_CHUNKS = NUM_NEW // 128
_ROWS_PER_BLOCK = 128


def _sc_stage1_body(num_tokens_hbm, sids_hbm, cnt_out_hbm, idx_out_hbm,
                    sids_v, cnt_v, idx_v):
  @pl.when((lax.axis_index("c") == 0) & (lax.axis_index("s") == 0))
  def _():
    pltpu.sync_copy(sids_hbm, sids_v)
    pltpu.sync_copy(num_tokens_hbm, cnt_v)

    @pl.loop(0, _GROUPS)
    def _(g):
      v = sids_v[g]
      r, is_last = plsc.scan_count(v)
      base = plsc.load_gather(cnt_v, [v])
      pos = base + r - 1
      plsc.store_scatter(cnt_v, [v], pos + 1, mask=is_last)
      flat = v * MAX_TOKENS + pos
      idx_v[g // 8, pl.ds((g % 8) * 16, 16)] = flat

    pltpu.sync_copy(cnt_v, cnt_out_hbm)
    pltpu.sync_copy(idx_v, idx_out_hbm)


def _sc_stage1(num_tokens, sids):
  mesh = plsc.VectorSubcoreMesh(core_axis_name="c", subcore_axis_name="s")
  return pl.kernel(
      _sc_stage1_body,
      out_type=(
          jax.ShapeDtypeStruct((MAX_SEQS,), jnp.int32),
          jax.ShapeDtypeStruct((_CHUNKS, 128), jnp.int32),
      ),
      mesh=mesh,
      compiler_params=pltpu.CompilerParams(needs_layout_passes=False),
      scratch_types=[
          pltpu.VMEM((_GROUPS, 16), jnp.int32),
          pltpu.VMEM((MAX_SEQS,), jnp.int32),
          pltpu.VMEM((_CHUNKS, 128), jnp.int32),
      ],
  )(num_tokens, sids)


def _tc_copy_body(tok_in, lp_in, tok_out, lp_out):
  tok_out[...] = tok_in[...]
  lp_out[...] = lp_in[...]


def _tc_copy(tokens, logprobs):
  n = MAX_SEQS * MAX_TOKENS
  nblk = MAX_SEQS // _ROWS_PER_BLOCK
  blk = n // nblk
  spec = pl.BlockSpec((blk,), lambda i: (i,))
  return pl.pallas_call(
      _tc_copy_body,
      grid=(nblk,),
      in_specs=[spec, spec],
      out_specs=[spec, spec],
      out_shape=(
          jax.ShapeDtypeStruct((n,), jnp.int32),
          jax.ShapeDtypeStruct((n,), jnp.float32),
      ),
      compiler_params=pltpu.CompilerParams(
          dimension_semantics=("arbitrary",),
      ),
  )(tokens.reshape(-1), logprobs.reshape(-1))


def _sc_stage2_body(idx_hbm, ntok_hbm, nlp_hbm, tok_flat, lp_flat,
                    idx_v, val_i, val_f, sem):
  cid = lax.axis_index("c")
  sid = lax.axis_index("s")

  @pl.when(sid < _CHUNKS)
  def _():
    pltpu.sync_copy(idx_hbm.at[sid], idx_v)

    @pl.when(cid == 0)
    def _():
      pltpu.sync_copy(ntok_hbm.at[sid], val_i)
      pltpu.async_copy(val_i, tok_flat.at[idx_v], sem).wait()

    @pl.when(cid == 1)
    def _():
      pltpu.sync_copy(nlp_hbm.at[sid], val_f)
      pltpu.async_copy(val_f, lp_flat.at[idx_v], sem).wait()


def _sc_stage2(idx, ntok, nlp, tok_ref, lp_ref):
  mesh = plsc.VectorSubcoreMesh(core_axis_name="c", subcore_axis_name="s")
  return pl.kernel(
      _sc_stage2_body,
      out_type=(),
      mesh=mesh,
      compiler_params=pltpu.CompilerParams(needs_layout_passes=False),
      scratch_types=[
          pltpu.VMEM((128,), jnp.int32),
          pltpu.VMEM((128,), jnp.int32),
          pltpu.VMEM((128,), jnp.float32),
          pltpu.SemaphoreType.DMA,
      ],
  )(idx, ntok, nlp, tok_ref, lp_ref)


@jax.jit
def _kernel_impl(tokens, logprobs, num_tokens, local_seq_ids, new_tokens,
                 new_log_probs):
  cnt, idx = _sc_stage1(num_tokens, local_seq_ids.reshape(_GROUPS, 16))
  tok_c, lp_c = _tc_copy(tokens, logprobs)
  tok_ref = jax.new_ref(tok_c)
  lp_ref = jax.new_ref(lp_c)
  _sc_stage2(idx, new_tokens.reshape(_CHUNKS, 128),
             new_log_probs.reshape(_CHUNKS, 128), tok_ref, lp_ref)
  return (
      tok_ref[...].reshape(MAX_SEQS, MAX_TOKENS),
      lp_ref[...].reshape(MAX_SEQS, MAX_TOKENS),
      cnt,
  )


def kernel(tokens, logprobs, num_tokens, local_seq_ids, new_tokens,
           new_log_probs, num_new_tokens):
  del num_new_tokens
  return _kernel_impl(tokens, logprobs, num_tokens, local_seq_ids, new_tokens,
                      new_log_probs)
